# Initial kernel scaffold; baseline (speedup 1.0000x reference)
import jax
import jax.numpy as jnp
from jax.experimental import pallas as pl


def kernel(feats, edge_index, ntypes, etypes, W_node, b_node, W_edge, b_edge):
    raise NotImplementedError("write your pallas kernel here")


# speedup vs baseline: 5.4887x; 5.4887x over previous
import jax
import jax.numpy as jnp
from jax import lax
from jax.experimental import pallas as pl
from jax.experimental.pallas import tpu as pltpu
from jax.experimental.pallas import tpu_sc as plsc

N = 10000
E = 320000
D = 128
DH = D // 2
NRELS = 8
NTYPES = 4

NC = 2
NS = 16

EP = 327680
ET = EP // NS
C1 = 1024
C2 = 128
NTRASH = 8
NPK = N * NRELS


def _k1_body(f_ref, w_ref, y_ref):
    y_ref[...] = jnp.dot(f_ref[...], w_ref[0], preferred_element_type=jnp.float32)


def _edge_transform(feats, W_edge):
    BN = 2000
    NB = N // BN
    return pl.pallas_call(
        _k1_body,
        grid=(NRELS, NB),
        in_specs=[
            pl.BlockSpec((BN, D), lambda r, i: (i, 0)),
            pl.BlockSpec((1, D, D), lambda r, i: (r, 0, 0)),
        ],
        out_specs=pl.BlockSpec((BN, D), lambda r, i: (r * NB + i, 0)),
        out_shape=jax.ShapeDtypeStruct((NRELS * N, D), jnp.float32),
    )(feats, W_edge)


def _k2_body(src_hbm, dst_hbm, et_hbm, y_hbm,
             agg_hbm, cnt_hbm,
             dstb1, etb1, onesb, pkb1,
             srcb, dstb, etb, cntb, scaleb, k2b, pkb, dsb, rowsb, rowshb,
             zb1,
             counts_sp, agg_sp, sem):
    c = lax.axis_index("c")
    s = lax.axis_index("s")

    def fill_z1(i, _):
        zb1[pl.ds(i * 16, 16)] = jnp.zeros((16,), jnp.float32)
        return 0

    lax.fori_loop(0, 5008 // 16, fill_z1, 0)

    def fill_z2(i, _):
        for j in range(DH // 16):
            rowshb[i, pl.ds(j * 16, 16)] = jnp.zeros((16,), jnp.float32)
        return 0

    lax.fori_loop(0, C2, fill_z2, 0)

    def fill_ones(i, _):
        onesb[pl.ds(i * 16, 16)] = jnp.ones((16,), jnp.float32)
        return 0

    lax.fori_loop(0, C1 // 16, fill_ones, 0)

    @pl.when(s < NS - 1)
    def _():
        pltpu.sync_copy(zb1, counts_sp.at[pl.ds(s * 5008, 5008)])
        for j in range(4):
            pltpu.sync_copy(rowshb, agg_sp.at[pl.ds(s * 624 + j * 128, 128)])
        pltpu.sync_copy(rowshb.at[pl.ds(0, 112)],
                        agg_sp.at[pl.ds(s * 624 + 512, 112)])

    @pl.when(s == NS - 1)
    def _():
        pltpu.sync_copy(zb1.at[pl.ds(0, 4896)],
                        counts_sp.at[pl.ds(15 * 5008, 4896)])
        for j in range(5):
            pltpu.sync_copy(rowshb, agg_sp.at[pl.ds(15 * 624 + j * 128, 128)])
        pltpu.sync_copy(rowshb.at[pl.ds(0, NTRASH)],
                        agg_sp.at[pl.ds(N, NTRASH)])

    plsc.subcore_barrier()

    def count_chunk(k, _):
        base = s * ET + k * C1
        pltpu.sync_copy(dst_hbm.at[pl.ds(base, C1)], dstb1)
        pltpu.sync_copy(et_hbm.at[pl.ds(base, C1)], etb1)

        def mk_pk(i, _):
            off = pl.ds(i * 16, 16)
            pkb1[off] = dstb1[off] * NRELS + etb1[off]
            return 0

        lax.fori_loop(0, C1 // 16, mk_pk, 0)
        pltpu.sync_copy(onesb, counts_sp.at[pkb1], add=True)
        return 0

    lax.fori_loop(0, ET // C1, count_chunk, 0)
    plsc.subcore_barrier()

    def edge_chunk(k, _):
        base = s * ET + k * C2
        pltpu.sync_copy(src_hbm.at[pl.ds(base, C2)], srcb)
        pltpu.sync_copy(dst_hbm.at[pl.ds(base, C2)], dstb)
        pltpu.sync_copy(et_hbm.at[pl.ds(base, C2)], etb)

        def mk_keys(i, _):
            off = pl.ds(i * 16, 16)
            et = etb[off]
            k2b[off] = et * N + srcb[off]
            pkb[off] = dstb[off] * NRELS + et
            dsb[off] = dstb[off]
            return 0

        lax.fori_loop(0, C2 // 16, mk_keys, 0)
        pltpu.async_copy(y_hbm.at[k2b], rowsb, sem).wait()
        pltpu.async_copy(counts_sp.at[pkb], cntb, sem).wait()

        def mk_scale(i, _):
            off = pl.ds(i * 16, 16)
            scaleb[off] = 1.0 / cntb[off]
            return 0

        lax.fori_loop(0, C2 // 16, mk_scale, 0)

        def scale_row(i, _):
            sv = plsc.load_gather(scaleb, [jnp.full((16,), i, jnp.int32)])
            for j in range(DH // 16):
                rowshb[i, pl.ds(j * 16, 16)] = (
                    rowsb[i, pl.ds(c * DH + j * 16, 16)] * sv)
            return 0

        lax.fori_loop(0, C2, scale_row, 0)
        pltpu.sync_copy(rowshb, agg_sp.at[dsb], add=True)
        return 0

    lax.fori_loop(0, ET // C2, edge_chunk, 0)
    plsc.subcore_barrier()

    @pl.when(s < NS - 1)
    def _():
        for off, nrows in ((0, 128), (128, 128), (256, 128), (384, 128), (512, 112)):
            sp = pl.ds(s * 624 + off, nrows)
            pltpu.sync_copy(agg_sp.at[sp], rowshb.at[pl.ds(0, nrows)])
            pltpu.sync_copy(rowshb.at[pl.ds(0, nrows)], agg_hbm.at[c, sp])

    @pl.when(s == NS - 1)
    def _():
        for off in (0, 128, 256, 384, 512):
            sp = pl.ds(15 * 624 + off, 128)
            pltpu.sync_copy(agg_sp.at[sp], rowshb)
            pltpu.sync_copy(rowshb, agg_hbm.at[c, sp])

    @pl.when(jnp.logical_and(c == 0, s < NS - 1))
    def _():
        cs = pl.ds(s * 5008, 5008)
        pltpu.sync_copy(counts_sp.at[cs], zb1)
        pltpu.sync_copy(zb1, cnt_hbm.at[cs])

    @pl.when(jnp.logical_and(c == 0, s == NS - 1))
    def _():
        cs = pl.ds(15 * 5008, 4880)
        pltpu.sync_copy(counts_sp.at[cs], zb1.at[pl.ds(0, 4880)])
        pltpu.sync_copy(zb1.at[pl.ds(0, 4880)], cnt_hbm.at[cs])


def _sc_aggregate(src, dst, etypes, y):
    mesh = plsc.VectorSubcoreMesh(core_axis_name="c", subcore_axis_name="s",
                                  num_cores=NC, num_subcores=NS)
    fn = pl.kernel(
        _k2_body,
        out_type=[
            jax.ShapeDtypeStruct((NC, N, DH), jnp.float32),
            jax.ShapeDtypeStruct((NPK,), jnp.float32),
        ],
        mesh=mesh,
        compiler_params=pltpu.CompilerParams(needs_layout_passes=False),
        scratch_types=[
            pltpu.VMEM((C1,), jnp.int32),
            pltpu.VMEM((C1,), jnp.int32),
            pltpu.VMEM((C1,), jnp.float32),
            pltpu.VMEM((C1,), jnp.int32),
            pltpu.VMEM((C2,), jnp.int32),
            pltpu.VMEM((C2,), jnp.int32),
            pltpu.VMEM((C2,), jnp.int32),
            pltpu.VMEM((C2,), jnp.float32),
            pltpu.VMEM((C2,), jnp.float32),
            pltpu.VMEM((C2,), jnp.int32),
            pltpu.VMEM((C2,), jnp.int32),
            pltpu.VMEM((C2,), jnp.int32),
            pltpu.VMEM((C2, D), jnp.float32),
            pltpu.VMEM((C2, DH), jnp.float32),
            pltpu.VMEM((5008,), jnp.float32),
            pltpu.VMEM_SHARED((NPK + 16, ), jnp.float32),
            pltpu.VMEM_SHARED((N + NTRASH, DH), jnp.float32),
            pltpu.SemaphoreType.DMA,
        ],
    )
    return fn(src, dst, etypes, y)


def _k3_body(f_ref, oh_ref, wn_ref, bn_ref, agg_ref, cnt_ref, be_ref, o_ref):
    f = f_ref[...]
    oh = oh_ref[...]
    acc = jnp.concatenate([agg_ref[0], agg_ref[1]], axis=-1)
    acc += oh @ bn_ref[...]
    nz = (cnt_ref[...] > 0.0).astype(jnp.float32)
    acc += nz @ be_ref[...]
    for t in range(NTYPES):
        acc += oh[:, t:t + 1] * jnp.dot(f, wn_ref[t], preferred_element_type=jnp.float32)
    o_ref[...] = jnp.maximum(acc, 0.0)


def _combine(feats, onehot, W_node, b_node, aggp, cnt, b_edge):
    BN = 2000
    NB = N // BN
    return pl.pallas_call(
        _k3_body,
        grid=(NB,),
        in_specs=[
            pl.BlockSpec((BN, D), lambda i: (i, 0)),
            pl.BlockSpec((BN, NTYPES), lambda i: (i, 0)),
            pl.BlockSpec((NTYPES, D, D), lambda i: (0, 0, 0)),
            pl.BlockSpec((NTYPES, D), lambda i: (0, 0)),
            pl.BlockSpec((2, BN, DH), lambda i: (0, i, 0)),
            pl.BlockSpec((BN, NRELS), lambda i: (i, 0)),
            pl.BlockSpec((NRELS, D), lambda i: (0, 0)),
        ],
        out_specs=pl.BlockSpec((BN, D), lambda i: (i, 0)),
        out_shape=jax.ShapeDtypeStruct((N, D), jnp.float32),
    )(feats, onehot, W_node, b_node, aggp, cnt, b_edge)


@jax.jit
def kernel(feats, edge_index, ntypes, etypes, W_node, b_node, W_edge, b_edge):
    npad = EP - E
    src = jnp.concatenate([edge_index[0], jnp.zeros((npad,), jnp.int32)])
    dst = jnp.concatenate([edge_index[1], jnp.full((npad,), N, jnp.int32)])
    etp = jnp.concatenate([etypes, jnp.zeros((npad,), jnp.int32)])
    y = _edge_transform(feats, W_edge)
    aggp, counts = _sc_aggregate(src, dst, etp, y)
    onehot = jax.nn.one_hot(ntypes, NTYPES, dtype=jnp.float32)
    cnt2d = counts.reshape(N, NRELS)
    return _combine(feats, onehot, W_node, b_node, aggp, cnt2d, b_edge)


# speedup vs baseline: 5.9130x; 1.0773x over previous
import jax
import jax.numpy as jnp
from jax import lax
from jax.experimental import pallas as pl
from jax.experimental.pallas import tpu as pltpu
from jax.experimental.pallas import tpu_sc as plsc

N = 10000
E = 320000
D = 128
DH = D // 2
NRELS = 8
NTYPES = 4

NC = 2
NS = 16

EP = 327680
ET = EP // NS
C1 = 1024
C2 = 128
NTRASH = 8
NPK = N * NRELS


def _k1_body(f_ref, w_ref, y_ref):
    y_ref[...] = jnp.dot(f_ref[...], w_ref[0], preferred_element_type=jnp.float32)


def _edge_transform(feats, W_edge):
    BN = 2000
    NB = N // BN
    return pl.pallas_call(
        _k1_body,
        grid=(NRELS, NB),
        in_specs=[
            pl.BlockSpec((BN, D), lambda r, i: (i, 0)),
            pl.BlockSpec((1, D, D), lambda r, i: (r, 0, 0)),
        ],
        out_specs=pl.BlockSpec((BN, D), lambda r, i: (r * NB + i, 0)),
        out_shape=jax.ShapeDtypeStruct((NRELS * N, D), jnp.float32),
    )(feats, W_edge)


def _k2_body(epk_hbm, y_hbm,
             agg_hbm, cnt_hbm,
             eb1, onesb, pkb1,
             eb2, scaleb, k2b, pkb, dsb, rowsb, rowshb,
             zb1,
             counts_sp, agg_sp, sem, sem2):
    c = lax.axis_index("c")
    s = lax.axis_index("s")

    def fill_z1(i, _):
        zb1[pl.ds(i * 16, 16)] = jnp.zeros((16,), jnp.float32)
        return 0

    lax.fori_loop(0, 5008 // 16, fill_z1, 0)

    def fill_z2(i, _):
        for j in range(DH // 16):
            rowshb[i, pl.ds(j * 16, 16)] = jnp.zeros((16,), jnp.float32)
        return 0

    lax.fori_loop(0, C2, fill_z2, 0)

    def fill_ones(i, _):
        onesb[pl.ds(i * 16, 16)] = jnp.ones((16,), jnp.float32)
        return 0

    lax.fori_loop(0, C1 // 16, fill_ones, 0)

    @pl.when(s < NS - 1)
    def _():
        pltpu.sync_copy(zb1, counts_sp.at[pl.ds(s * 5008, 5008)])
        for j in range(4):
            pltpu.sync_copy(rowshb, agg_sp.at[pl.ds(s * 624 + j * 128, 128)])
        pltpu.sync_copy(rowshb.at[pl.ds(0, 112)],
                        agg_sp.at[pl.ds(s * 624 + 512, 112)])

    @pl.when(s == NS - 1)
    def _():
        pltpu.sync_copy(zb1.at[pl.ds(0, 4896)],
                        counts_sp.at[pl.ds(15 * 5008, 4896)])
        for j in range(5):
            pltpu.sync_copy(rowshb, agg_sp.at[pl.ds(15 * 624 + j * 128, 128)])
        pltpu.sync_copy(rowshb.at[pl.ds(0, NTRASH)],
                        agg_sp.at[pl.ds(N, NTRASH)])

    plsc.subcore_barrier()

    def count_chunk(k, _):
        base = (s * ET + k * C1) * 3
        pltpu.sync_copy(epk_hbm.at[pl.ds(base, C1 * 3)], eb1)

        def mk_pk(i, _):
            blk = (i // 8) * 384
            off = pl.ds(blk + 128 + (i % 8) * 16, 16)
            off2 = pl.ds(blk + 256 + (i % 8) * 16, 16)
            pkb1[pl.ds(i * 16, 16)] = eb1[off] * NRELS + eb1[off2]
            return 0

        lax.fori_loop(0, C1 // 16, mk_pk, 0)
        pltpu.sync_copy(onesb, counts_sp.at[pkb1], add=True)
        return 0

    lax.fori_loop(0, ET // C1, count_chunk, 0)
    plsc.subcore_barrier()

    @pl.when(s < NS - 1)
    def _():
        cs = pl.ds(s * 5008, 5008)
        pltpu.sync_copy(counts_sp.at[cs], zb1)

        def recip(i, _):
            off = pl.ds(i * 16, 16)
            zb1[off] = 1.0 / zb1[off]
            return 0

        lax.fori_loop(0, 5008 // 16, recip, 0)
        pltpu.sync_copy(zb1, counts_sp.at[cs])

    @pl.when(s == NS - 1)
    def _():
        cs = pl.ds(15 * 5008, 4896)
        zs = pl.ds(0, 4896)
        pltpu.sync_copy(counts_sp.at[cs], zb1.at[zs])

        def recip(i, _):
            off = pl.ds(i * 16, 16)
            zb1[off] = 1.0 / zb1[off]
            return 0

        lax.fori_loop(0, 4896 // 16, recip, 0)
        pltpu.sync_copy(zb1.at[zs], counts_sp.at[cs])
    plsc.subcore_barrier()

    def edge_chunk(k, _):
        base = (s * ET + k * C2) * 3
        pltpu.sync_copy(epk_hbm.at[pl.ds(base, C2 * 3)], eb2)

        def mk_keys(i, _):
            off = pl.ds(i * 16, 16)
            off2 = pl.ds(128 + i * 16, 16)
            off3 = pl.ds(256 + i * 16, 16)
            et = eb2[off3]
            k2b[off] = et * N + eb2[off]
            pkb[off] = eb2[off2] * NRELS + et
            dsb[off] = eb2[off2]
            return 0

        lax.fori_loop(0, C2 // 16, mk_keys, 0)
        cg = pltpu.async_copy(counts_sp.at[pkb], scaleb, sem2)
        pltpu.async_copy(y_hbm.at[k2b], rowsb, sem).wait()
        cg.wait()

        def scale_row(i, _):
            sv = plsc.load_gather(scaleb, [jnp.full((16,), i, jnp.int32)])
            for j in range(DH // 16):
                rowshb[i, pl.ds(j * 16, 16)] = (
                    rowsb[i, pl.ds(c * DH + j * 16, 16)] * sv)
            return 0

        lax.fori_loop(0, C2, scale_row, 0, unroll=4)
        pltpu.sync_copy(rowshb, agg_sp.at[dsb], add=True)
        return 0

    lax.fori_loop(0, ET // C2, edge_chunk, 0)
    plsc.subcore_barrier()

    @pl.when(s < NS - 1)
    def _():
        for off, nrows in ((0, 128), (128, 128), (256, 128), (384, 128), (512, 112)):
            sp = pl.ds(s * 624 + off, nrows)
            pltpu.sync_copy(agg_sp.at[sp], rowshb.at[pl.ds(0, nrows)])
            pltpu.sync_copy(rowshb.at[pl.ds(0, nrows)], agg_hbm.at[c, sp])

    @pl.when(s == NS - 1)
    def _():
        for off in (0, 128, 256, 384, 512):
            sp = pl.ds(15 * 624 + off, 128)
            pltpu.sync_copy(agg_sp.at[sp], rowshb)
            pltpu.sync_copy(rowshb, agg_hbm.at[c, sp])

    @pl.when(jnp.logical_and(c == 0, s < NS - 1))
    def _():
        cs = pl.ds(s * 5008, 5008)
        pltpu.sync_copy(counts_sp.at[cs], zb1)
        pltpu.sync_copy(zb1, cnt_hbm.at[cs])

    @pl.when(jnp.logical_and(c == 0, s == NS - 1))
    def _():
        cs = pl.ds(15 * 5008, 4880)
        pltpu.sync_copy(counts_sp.at[cs], zb1.at[pl.ds(0, 4880)])
        pltpu.sync_copy(zb1.at[pl.ds(0, 4880)], cnt_hbm.at[cs])


def _sc_aggregate(epacked, y):
    mesh = plsc.VectorSubcoreMesh(core_axis_name="c", subcore_axis_name="s",
                                  num_cores=NC, num_subcores=NS)
    fn = pl.kernel(
        _k2_body,
        out_type=[
            jax.ShapeDtypeStruct((NC, N, DH), jnp.float32),
            jax.ShapeDtypeStruct((NPK,), jnp.float32),
        ],
        mesh=mesh,
        compiler_params=pltpu.CompilerParams(needs_layout_passes=False),
        scratch_types=[
            pltpu.VMEM((C1 * 3,), jnp.int32),
            pltpu.VMEM((C1,), jnp.float32),
            pltpu.VMEM((C1,), jnp.int32),
            pltpu.VMEM((C2 * 3,), jnp.int32),
            pltpu.VMEM((C2,), jnp.float32),
            pltpu.VMEM((C2,), jnp.int32),
            pltpu.VMEM((C2,), jnp.int32),
            pltpu.VMEM((C2,), jnp.int32),
            pltpu.VMEM((C2, D), jnp.float32),
            pltpu.VMEM((C2, DH), jnp.float32),
            pltpu.VMEM((5008,), jnp.float32),
            pltpu.VMEM_SHARED((NPK + 16, ), jnp.float32),
            pltpu.VMEM_SHARED((N + NTRASH, DH), jnp.float32),
            pltpu.SemaphoreType.DMA,
            pltpu.SemaphoreType.DMA,
        ],
    )
    return fn(epacked, y)


def _k3_body(f_ref, oh_ref, wn_ref, bn_ref, agg_ref, cnt_ref, be_ref, o_ref):
    f = f_ref[...]
    oh = oh_ref[...]
    acc = jnp.concatenate([agg_ref[0], agg_ref[1]], axis=-1)
    acc += oh @ bn_ref[...]
    nz = (cnt_ref[...] > 0.0).astype(jnp.float32)
    acc += nz @ be_ref[...]
    for t in range(NTYPES):
        acc += oh[:, t:t + 1] * jnp.dot(f, wn_ref[t], preferred_element_type=jnp.float32)
    o_ref[...] = jnp.maximum(acc, 0.0)


def _combine(feats, onehot, W_node, b_node, aggp, cnt, b_edge):
    BN = 2000
    NB = N // BN
    return pl.pallas_call(
        _k3_body,
        grid=(NB,),
        in_specs=[
            pl.BlockSpec((BN, D), lambda i: (i, 0)),
            pl.BlockSpec((BN, NTYPES), lambda i: (i, 0)),
            pl.BlockSpec((NTYPES, D, D), lambda i: (0, 0, 0)),
            pl.BlockSpec((NTYPES, D), lambda i: (0, 0)),
            pl.BlockSpec((2, BN, DH), lambda i: (0, i, 0)),
            pl.BlockSpec((BN, NRELS), lambda i: (i, 0)),
            pl.BlockSpec((NRELS, D), lambda i: (0, 0)),
        ],
        out_specs=pl.BlockSpec((BN, D), lambda i: (i, 0)),
        out_shape=jax.ShapeDtypeStruct((N, D), jnp.float32),
    )(feats, onehot, W_node, b_node, aggp, cnt, b_edge)


@jax.jit
def kernel(feats, edge_index, ntypes, etypes, W_node, b_node, W_edge, b_edge):
    npad = EP - E
    src = jnp.concatenate([edge_index[0], jnp.zeros((npad,), jnp.int32)])
    dst = jnp.concatenate([edge_index[1], jnp.full((npad,), N, jnp.int32)])
    etp = jnp.concatenate([etypes, jnp.zeros((npad,), jnp.int32)])
    epacked = jnp.stack([src.reshape(-1, 128), dst.reshape(-1, 128),
                         etp.reshape(-1, 128)], axis=1).reshape(-1)
    y = _edge_transform(feats, W_edge)
    aggp, counts = _sc_aggregate(epacked, y)
    onehot = jax.nn.one_hot(ntypes, NTYPES, dtype=jnp.float32)
    cnt2d = counts.reshape(N, NRELS)
    return _combine(feats, onehot, W_node, b_node, aggp, cnt2d, b_edge)
